# baseline (speedup 1.0000x reference)
import functools

import jax
import jax.numpy as jnp
from jax import lax
from jax.experimental import pallas as pl
from jax.experimental.pallas import tpu as pltpu
from jax.experimental.pallas import tpu_sc as plsc

N = 10000
E = 160000
D_IN = 256
D_OUT = 256
H = 128
CH = 32
NC = 2
NS = 16
CPT = 320
NCHUNK_PAD = NS * CPT
E_PAD = NCHUNK_PAD * CH
SLOTS = 8
GLA = 4
ILA = 6
SDR = 2
NPAD = 10016


_BR = 400


def _mm_body(x_ref, w_ref, o_ref):
    o_ref[...] = jnp.dot(x_ref[...], w_ref[...],
                         preferred_element_type=jnp.float32)


def _matmul(inputs, W):
    grid = (N // _BR, NC)
    return pl.pallas_call(
        _mm_body,
        grid=grid,
        in_specs=[
            pl.BlockSpec((_BR, D_IN), lambda i, j: (i, 0)),
            pl.BlockSpec((D_IN, H), lambda i, j: (0, j)),
        ],
        out_specs=pl.BlockSpec((_BR, H), lambda i, j: (j * (N // _BR) + i, 0)),
        out_shape=jax.ShapeDtypeStruct((NC * N, H), jnp.float32),
    )(inputs, W)


def _sc_body(xcat_hbm, src_hbm, dst_hbm, adj_hbm, out_hbm,
             src4, dst4, adj4, rows, acc, *sems):
    c = lax.axis_index("c")
    s = lax.axis_index("s")
    gsems = sems[:SLOTS]
    ssems = sems[SLOTS:2 * SLOTS]
    isems = sems[2 * SLOTS:]
    cN = c * N
    base_chunk = s * CPT

    zero16 = jnp.zeros((16,), jnp.float32)

    def zrow(i, carry):
        for j in range(H // 16):
            rows[0, i, pl.ds(j * 16, 16)] = zero16
        return carry

    lax.fori_loop(0, CH, zrow, 0)

    @pl.when(s < NS - 1)
    def _():
        for k in range(640 // CH):
            pltpu.sync_copy(rows.at[0], acc.at[pl.ds(s * 640 + k * CH, CH)])

    @pl.when(s == NS - 1)
    def _():
        for k in range(416 // CH):
            pltpu.sync_copy(rows.at[0], acc.at[pl.ds(9600 + k * CH, CH)])

    plsc.subcore_barrier()

    def idx_copies(k, t):
        eb = (base_chunk + k) * CH
        return (
            pltpu.make_async_copy(src_hbm.at[pl.ds(eb, CH)],
                                  src4.at[pl.ds(t * CH, CH)], isems[t]),
            pltpu.make_async_copy(dst_hbm.at[pl.ds(eb, CH)], dst4.at[t],
                                  isems[t]),
            pltpu.make_async_copy(adj_hbm.at[pl.ds(eb, CH)],
                                  adj4.at[pl.ds(t * CH, CH)], isems[t]),
        )

    def idx_start(k, t):
        for cp in idx_copies(k, t):
            cp.start()

    def idx_wait(k, t):
        for cp in idx_copies(k, t):
            cp.wait()

    def fold(t):
        for g in range(CH // 16):
            sl = pl.ds(t * CH + g * 16, 16)
            src4[sl] = src4[sl] + cN

    def gather_copy(t):
        return pltpu.make_async_copy(xcat_hbm.at[src4.at[pl.ds(t * CH, CH)]], rows.at[t],
                                     gsems[t])

    def scatter_copy(t):
        return pltpu.make_async_copy(rows.at[t], acc.at[dst4.at[t]],
                                     ssems[t])

    def scatter_start(t):
        pltpu.async_copy(rows.at[t], acc.at[dst4.at[t]], ssems[t], add=True)

    dims = lax.GatherDimensionNumbers(
        offset_dims=(), collapsed_slice_dims=(0,), start_index_map=(0,))

    def scale_chunk(t):
        def escale(e, cc):
            a16 = adj4[pl.ds(t * CH + (e & ~15), 16)]
            ae = lax.gather(
                a16, jnp.broadcast_to(e & 15, (16,))[:, None], dims, (1,),
                mode=lax.GatherScatterMode.PROMISE_IN_BOUNDS)
            for v in range(H // 16):
                sl = pl.ds(v * 16, 16)
                rows[t, e, sl] = rows[t, e, sl] * ae
            return cc

        lax.fori_loop(0, CH, escale, 0)

    for k in range(ILA):
        idx_start(k, k % SLOTS)
    for k in range(GLA):
        idx_wait(k, k % SLOTS)
        fold(k % SLOTS)
        gather_copy(k % SLOTS).start()

    def super_step(q, carry):
        for jj in range(SLOTS):
            j = q * SLOTS + jj

            live = j - SDR < CPT
            if jj >= SDR:
                @pl.when(live)
                def _():
                    scatter_copy((jj - SDR) % SLOTS).wait()
            else:
                @pl.when(live & (q > 0))
                def _():
                    scatter_copy((jj - SDR) % SLOTS).wait()

            @pl.when(j + GLA < CPT)
            def _():
                t = (jj + GLA) % SLOTS
                idx_wait(j + GLA, t)
                fold(t)
                gather_copy(t).start()

            @pl.when(j + ILA < CPT)
            def _():
                idx_start(j + ILA, (jj + ILA) % SLOTS)

            @pl.when(j < CPT)
            def _():
                gather_copy(jj).wait()
                scale_chunk(jj)
                scatter_start(jj)

        return carry

    lax.fori_loop(0, -(-(CPT + SDR) // SLOTS), super_step, 0)

    plsc.subcore_barrier()

    @pl.when(s < NS - 1)
    def _():
        pltpu.sync_copy(acc.at[pl.ds(s * 640, 640)],
                        out_hbm.at[c, pl.ds(s * 640, 640)])

    @pl.when(s == NS - 1)
    def _():
        pltpu.sync_copy(acc.at[pl.ds(9600, 416)],
                        out_hbm.at[c, pl.ds(9600, 416)])


def _sc_scatter(x_cat, edge_index, adj_vals):
    mesh = plsc.VectorSubcoreMesh(core_axis_name="c", subcore_axis_name="s")
    fn = functools.partial(
        pl.kernel,
        out_type=jax.ShapeDtypeStruct((NC, NPAD, H), jnp.float32),
        mesh=mesh,
        scratch_types=[
            pltpu.VMEM((SLOTS * CH,), jnp.int32),
            pltpu.VMEM((SLOTS, CH), jnp.int32),
            pltpu.VMEM((SLOTS * CH,), jnp.float32),
            pltpu.VMEM((SLOTS, CH, H), jnp.float32),
            pltpu.VMEM_SHARED((NPAD, H), jnp.float32),
        ] + [pltpu.SemaphoreType.DMA] * (3 * SLOTS),
    )(_sc_body)
    pad = E_PAD - E
    src_e = jnp.pad(edge_index[1], (0, pad))
    dst_e = jnp.pad(edge_index[0], (0, pad))
    adj_e = jnp.pad(adj_vals, (0, pad))
    return fn(x_cat, src_e, dst_e, adj_e)


_BR2 = 400


def _epi_body(y_ref, b_ref, o_ref):
    y = jnp.concatenate([y_ref[0], y_ref[1]], axis=1) + b_ref[...][None, :]
    y = jnp.maximum(y, 0.0)
    nrm = jnp.sqrt(jnp.sum(y * y, axis=1, keepdims=True))
    o_ref[...] = y / jnp.maximum(nrm, 1e-12)


def _epilogue(y_cat, b):
    return pl.pallas_call(
        _epi_body,
        grid=(N // _BR2,),
        in_specs=[
            pl.BlockSpec((NC, _BR2, H), lambda i: (0, i, 0)),
            pl.BlockSpec((D_OUT,), lambda i: (0,)),
        ],
        out_specs=pl.BlockSpec((_BR2, D_OUT), lambda i: (i, 0)),
        out_shape=jax.ShapeDtypeStruct((N, D_OUT), jnp.float32),
    )(y_cat, b)


def kernel(inputs, edge_index, adj_vals, W, b):
    x_cat = _matmul(inputs, W)
    y_cat = _sc_scatter(x_cat, edge_index, adj_vals)
    return _epilogue(y_cat, b)
